# baseline (speedup 1.0000x reference)
import functools
import jax
import jax.numpy as jnp
from jax import lax
from jax.experimental import pallas as pl
from jax.experimental.pallas import tpu as pltpu, tpu_sc as plsc

_A = 8
_NA = 16
_D = 128
_OUT = 64


def _tc_body(h_ref, pi_ref, act_ref,
             kW1_ref, kb1_ref, kW2_ref, kb2_ref,
             qW1_ref, qb1_ref, qW2_ref, qb2_ref,
             zz_ref, w_ref):
    EB = h_ref.shape[0]
    R = EB * _A
    hb = h_ref[...].reshape(R, _D)
    key = jnp.tanh(
        jnp.dot(hb, kW1_ref[...], preferred_element_type=jnp.float32)
        + kb1_ref[...])
    key = (jnp.dot(key, kW2_ref[...], preferred_element_type=jnp.float32)
           + kb2_ref[...]).reshape(EB, _A, _OUT)
    qry = jnp.tanh(
        jnp.dot(hb, qW1_ref[...], preferred_element_type=jnp.float32)
        + qb1_ref[...])
    qry = (jnp.dot(qry, qW2_ref[...], preferred_element_type=jnp.float32)
           + qb2_ref[...]).reshape(EB, _A, _OUT)
    s = jnp.sum(qry[:, :, None, :] * key[:, None, :, :], axis=-1)
    w = jax.nn.sigmoid(s * 0.125)
    pa2 = pi_ref[...].reshape(EB, _A * _NA)
    da2 = (act_ref[...] - pi_ref[...]).reshape(EB, _A * _NA)
    w2 = jnp.repeat(w, _NA, axis=2)
    wd = w2 * da2[:, None, :]
    z2 = wd + pa2[:, None, :]
    lane = lax.broadcasted_iota(jnp.int32, (_A * _NA, _NA), 0) % _NA
    col = lax.broadcasted_iota(jnp.int32, (_A * _NA, _NA), 1)
    G = (lane == col).astype(jnp.float32)
    S = jnp.dot(z2.reshape(R, _A * _NA), G,
                preferred_element_type=jnp.float32)
    S2 = jnp.tile(S, (1, _A)).reshape(EB, _A, _A * _NA)
    zz_ref[...] = (S2 - wd) * 0.125
    w_ref[...] = w


def _tc_compute(h3, pi3, act3, kW1, kb1, kW2, kb2, qW1, qb1, qW2, qb2):
    E = h3.shape[0]
    EB = 256
    grid = (E // EB,)

    def blk(shape):
        return pl.BlockSpec(shape, lambda b: (b,) + (0,) * (len(shape) - 1))

    def full(shape):
        return pl.BlockSpec(shape, lambda b: (0,) * len(shape))

    return pl.pallas_call(
        _tc_body,
        grid=grid,
        in_specs=[
            blk((EB, _A, _D)),
            blk((EB, _A, _NA)),
            blk((EB, _A, _NA)),
            full((_D, 32)), full((1, 32)), full((32, _OUT)), full((1, _OUT)),
            full((_D, 32)), full((1, 32)), full((32, _OUT)), full((1, _OUT)),
        ],
        out_specs=[
            blk((EB, _A, _A * _NA)),
            blk((EB, _A, _A)),
        ],
        out_shape=[
            jax.ShapeDtypeStruct((E, _A, _A * _NA), jnp.float32),
            jax.ShapeDtypeStruct((E, _A, _A), jnp.float32),
        ],
    )(h3, pi3, act3,
      kW1, kb1.reshape(1, 32), kW2, kb2.reshape(1, _OUT),
      qW1, qb1.reshape(1, 32), qW2, qb2.reshape(1, _OUT))


_EPB = 4


def _sc_assemble(obs_proc, zzc):
    N = obs_proc.shape[0]
    E = N // _A
    info = plsc.get_sparse_core_info()
    nw = info.num_cores * info.num_subcores
    envs_per_w = E // nw
    n_batches = envs_per_w // _EPB
    mesh = plsc.VectorSubcoreMesh(core_axis_name="c", subcore_axis_name="s")

    @functools.partial(
        pl.kernel, mesh=mesh,
        out_type=jax.ShapeDtypeStruct((N, _A, _D + _NA), jnp.float32),
        scratch_types=[
            pltpu.VMEM((_EPB * _A, _D), jnp.float32),
            pltpu.VMEM((_EPB, _A, _A * _NA), jnp.float32),
            pltpu.VMEM((_EPB * _A, _A, _D + _NA), jnp.float32),
        ],
    )
    def k(obs_hbm, zz_hbm, out_hbm, obs_v, zz_v, tmpl):
        wid = lax.axis_index("s") * info.num_cores + lax.axis_index("c")
        env0 = wid * envs_per_w

        def body(t, carry):
            e0 = env0 + t * _EPB
            pltpu.sync_copy(obs_hbm.at[pl.ds(e0 * _A, _EPB * _A)], obs_v)
            pltpu.sync_copy(zz_hbm.at[pl.ds(e0, _EPB)], zz_v)
            for b in range(_EPB):
                for j in range(_A):
                    for c in range(_D // 16):
                        v = obs_v[b * _A + j, pl.ds(c * 16, 16)]
                        for i in range(_A):
                            tmpl[b * _A + i, j, pl.ds(c * 16, 16)] = v
                    for i in range(_A):
                        tmpl[b * _A + i, j, pl.ds(_D, _NA)] = (
                            zz_v[b, i, pl.ds(j * _NA, _NA)])
            pltpu.sync_copy(tmpl, out_hbm.at[pl.ds(e0 * _A, _EPB * _A)])
            return carry

        lax.fori_loop(0, n_batches, body, 0)

    return k(obs_proc, zzc)


def kernel(h, policies, actions, obs_proc, edge_index,
           kW1, kb1, kW2, kb2, qW1, qb1, qW2, qb2):
    N = h.shape[0]
    E = N // _A
    h3 = h.reshape(E, _A, _D)
    pi3 = policies.reshape(E, _A, _NA)
    act3 = actions.reshape(E, _A, _NA)
    zzc, w = _tc_compute(h3, pi3, act3,
                         kW1, kb1, kW2, kb2, qW1, qb1, qW2, qb2)
    out = _sc_assemble(obs_proc, zzc)
    return out, w.reshape(N, _A, 1)


# speedup vs baseline: 2.1329x; 2.1329x over previous
import jax
import jax.numpy as jnp
from jax.experimental import pallas as pl

_A = 8
_NA = 16
_D = 128
_OUT = 64


def _body(h_ref, pi_ref, act_ref, obs_ref,
          kW1_ref, kb1_ref, kW2_ref, kb2_ref,
          qW1_ref, qb1_ref, qW2_ref, qb2_ref,
          out_ref, w_ref):
    EB = h_ref.shape[0]
    R = EB * _A
    hb = h_ref[...].reshape(R, _D)
    key = jnp.tanh(
        jnp.dot(hb, kW1_ref[...], preferred_element_type=jnp.float32)
        + kb1_ref[...])
    key = (jnp.dot(key, kW2_ref[...], preferred_element_type=jnp.float32)
           + kb2_ref[...]).reshape(EB, _A, _OUT)
    qry = jnp.tanh(
        jnp.dot(hb, qW1_ref[...], preferred_element_type=jnp.float32)
        + qb1_ref[...])
    qry = (jnp.dot(qry, qW2_ref[...], preferred_element_type=jnp.float32)
           + qb2_ref[...]).reshape(EB, _A, _OUT)
    s = jnp.sum(qry[:, :, None, :] * key[:, None, :, :], axis=-1)
    w = jax.nn.sigmoid(s * 0.125)
    pi = pi_ref[...]
    act = act_ref[...]
    pib = pi[:, None, :, :]
    z = w[..., None] * (act[:, None, :, :] - pib) + pib
    S = jnp.sum(z, axis=2)
    zz = (S[:, :, None, :] - z + pib) * 0.125
    obs = obs_ref[...]
    for i in range(_A):
        out_ref[:, i, :, 0:_D] = obs
    out_ref[:, :, :, _D:] = zz
    w_ref[...] = w


def kernel(h, policies, actions, obs_proc, edge_index,
           kW1, kb1, kW2, kb2, qW1, qb1, qW2, qb2):
    N = h.shape[0]
    E = N // _A
    EB = 256
    grid = (E // EB,)
    h3 = h.reshape(E, _A, _D)
    pi3 = policies.reshape(E, _A, _NA)
    act3 = actions.reshape(E, _A, _NA)
    obs3 = obs_proc.reshape(E, _A, _D)

    def blk(shape):
        return pl.BlockSpec(shape, lambda b: (b,) + (0,) * (len(shape) - 1))

    def full(shape):
        return pl.BlockSpec(shape, lambda b: (0,) * len(shape))

    out, w = pl.pallas_call(
        _body,
        grid=grid,
        in_specs=[
            blk((EB, _A, _D)),
            blk((EB, _A, _NA)),
            blk((EB, _A, _NA)),
            blk((EB, _A, _D)),
            full((_D, 32)), full((1, 32)), full((32, _OUT)), full((1, _OUT)),
            full((_D, 32)), full((1, 32)), full((32, _OUT)), full((1, _OUT)),
        ],
        out_specs=[
            blk((EB, _A, _A, _D + _NA)),
            blk((EB, _A, _A)),
        ],
        out_shape=[
            jax.ShapeDtypeStruct((E, _A, _A, _D + _NA), jnp.float32),
            jax.ShapeDtypeStruct((E, _A, _A), jnp.float32),
        ],
    )(h3, pi3, act3, obs3,
      kW1, kb1.reshape(1, 32), kW2, kb2.reshape(1, _OUT),
      qW1, qb1.reshape(1, 32), qW2, qb2.reshape(1, _OUT))
    return out.reshape(N, _A, _D + _NA), w.reshape(N, _A, 1)
